# baseline (speedup 1.0000x reference)
import functools

import jax
import jax.numpy as jnp
from jax import lax
from jax.experimental import pallas as pl
from jax.experimental.pallas import tpu as pltpu
from jax.experimental.pallas import tpu_sc as plsc

_H = 256
_NROWS = 160000
_NSEG = 10000
_R = 3200
_W = 256
_SB = 2000

_NSC = 2
_NSUB = 16
_B = 128
_NB = 79
_M = _NB * _B
_NPAD = _NSUB * _M
_HSEG = _NSEG // _NSC
_CPAD = 5008

_DN = (((1,), (1,)), ((), ()))


def _phi_reduce_body(idx_ref, x_ref, w1_ref, b1_ref, w2_ref, b2_ref, acc_ref):
    i = pl.program_id(0)

    @pl.when(i == 0)
    def _init():
        acc_ref[...] = jnp.zeros_like(acc_ref)

    _RH = _R // 2

    def phi(xh):
        h = jnp.maximum(
            lax.dot_general(xh.astype(jnp.bfloat16), w1_ref[...], _DN,
                            preferred_element_type=jnp.float32)
            .astype(jnp.bfloat16) + b1_ref[...], 0.0)
        return jnp.maximum(
            lax.dot_general(h, w2_ref[...], _DN,
                            preferred_element_type=jnp.float32)
            .astype(jnp.bfloat16) + b2_ref[...], 0.0)

    hb_a = phi(x_ref[:_RH, :])
    hb_b = phi(x_ref[_RH:, :])

    idx = idx_ref[0]

    def reduce_half(idx_h, hb, prev_end):
        s0 = jnp.min(idx_h)
        max_idx = jnp.max(idx_h)

        def accum(base, oh):
            contrib = jnp.dot(oh, hb, preferred_element_type=jnp.float32)
            acc_ref[pl.ds(base, _W), :] += contrib
            return base + _W

        base0 = jnp.minimum((s0 // 8) * 8, _NSEG - _W)
        lanes0 = lax.broadcasted_iota(jnp.int32, (_W, _RH), 0) + base0
        idx_b0 = jnp.broadcast_to(idx_h, (_W, _RH))
        s1 = accum(base0, (idx_b0 == lanes0).astype(jnp.bfloat16))

        n_extra = jnp.maximum(0, (max_idx - s1 + _W) // _W)

        def pass_body(_, s):
            base = jnp.minimum((s // 8) * 8, _NSEG - _W)
            lanes = lax.broadcasted_iota(jnp.int32, (_W, _RH), 0) + base
            idx_b = jnp.broadcast_to(idx_h, (_W, _RH))
            oh = ((idx_b == lanes) & (idx_b >= s)).astype(jnp.bfloat16)
            return accum(base, oh)

        return lax.fori_loop(0, n_extra, pass_body, s1)

    reduce_half(idx[:, :_RH], hb_a, 0)
    reduce_half(idx[:, _RH:], hb_b, 0)


def _sc_counts_body(idx_hbm, out_hbm, idx_vm, lidx_vm, ones_v, zbuf, cnt_sh,
                    sem):
    c = lax.axis_index("c")
    s = lax.axis_index("s")

    for k in range(_B // 16):
        ones_v[pl.ds(16 * k, 16)] = jnp.ones((16,), jnp.float32)

    @pl.when(s == 0)
    def _zero():
        def zb(j, carry):
            zbuf[pl.ds(16 * j, 16)] = jnp.zeros((16,), jnp.float32)
            return carry
        lax.fori_loop(0, _CPAD // 16, zb, 0)
        pltpu.sync_copy(zbuf, cnt_sh)

    plsc.subcore_barrier()

    pltpu.sync_copy(idx_hbm.at[s], idx_vm)

    lo = c * _HSEG

    def lrow(j, carry):
        for k in range(_B // 16):
            v = idx_vm[j, pl.ds(16 * k, 16)] - lo
            ok = (v >= 0) & (v < _HSEG)
            lidx_vm[j, pl.ds(16 * k, 16)] = jnp.where(ok, v, _HSEG)
        return carry

    lax.fori_loop(0, _NB, lrow, 0)

    def fire(j, carry):
        pltpu.async_copy(ones_v, cnt_sh.at[lidx_vm.at[j]], sem, add=True)
        return carry

    lax.fori_loop(0, _NB, fire, 0)

    def drain(j, carry):
        pltpu.make_async_copy(ones_v, cnt_sh.at[lidx_vm.at[0]], sem).wait()
        return carry

    lax.fori_loop(0, _NB, drain, 0)

    plsc.subcore_barrier()

    @pl.when(s == 0)
    def _flush():
        pltpu.sync_copy(cnt_sh, out_hbm.at[c])


def _sc_counts(idx3):
    mesh = plsc.VectorSubcoreMesh(core_axis_name="c", subcore_axis_name="s")
    f = functools.partial(
        pl.kernel,
        mesh=mesh,
        out_type=jax.ShapeDtypeStruct((_NSC, _CPAD), jnp.float32),
        scratch_types=[
            pltpu.VMEM((_NB, _B), jnp.int32),
            pltpu.VMEM((_NB, _B), jnp.int32),
            pltpu.VMEM((_B,), jnp.float32),
            pltpu.VMEM((_CPAD,), jnp.float32),
            pltpu.VMEM_SHARED((_CPAD,), jnp.float32),
            pltpu.SemaphoreType.DMA,
        ],
    )(_sc_counts_body)
    return f(idx3)


def _rho_body(acc_ref, cnt_ref, w3_ref, b3_ref, w4_ref, b4_ref, out_ref):
    pooled = acc_ref[...] / jnp.maximum(cnt_ref[...], 1.0)
    o = jnp.maximum(
        lax.dot_general(pooled.astype(jnp.bfloat16), w3_ref[...], _DN,
                        preferred_element_type=jnp.float32) + b3_ref[...], 0.0)
    o = jnp.maximum(
        lax.dot_general(o.astype(jnp.bfloat16), w4_ref[...], _DN,
                        preferred_element_type=jnp.float32) + b4_ref[...], 0.0)
    out_ref[...] = o


@jax.jit
def kernel(x, molecule_idx, W1, b1, W2, b2, W3, b3, W4, b4):
    nchunks = _NROWS // _R
    idx32 = molecule_idx.astype(jnp.int32)
    idx3 = idx32.reshape(nchunks, 1, _R)
    bf = jnp.bfloat16

    idx_pad = jnp.pad(idx32, (0, _NPAD - _NROWS), constant_values=_NSEG)
    cnt2 = _sc_counts(idx_pad.reshape(_NSUB, _NB, _B))
    cnt = jnp.concatenate([cnt2[0, :_HSEG], cnt2[1, :_HSEG]]).reshape(_NSEG, 1)

    acc = pl.pallas_call(
        _phi_reduce_body,
        grid=(nchunks,),
        in_specs=[
            pl.BlockSpec((1, 1, _R), lambda i: (i, 0, 0)),
            pl.BlockSpec((_R, _H), lambda i: (i, 0)),
            pl.BlockSpec((_H, _H), lambda i: (0, 0)),
            pl.BlockSpec((1, _H), lambda i: (0, 0)),
            pl.BlockSpec((_H, _H), lambda i: (0, 0)),
            pl.BlockSpec((1, _H), lambda i: (0, 0)),
        ],
        out_specs=pl.BlockSpec((_NSEG, _H), lambda i: (0, 0)),
        out_shape=jax.ShapeDtypeStruct((_NSEG, _H), jnp.float32),
    )(idx3, x, W1.astype(bf), b1.reshape(1, _H).astype(bf),
      W2.astype(bf), b2.reshape(1, _H).astype(bf))

    out = pl.pallas_call(
        _rho_body,
        grid=(_NSEG // _SB,),
        in_specs=[
            pl.BlockSpec((_SB, _H), lambda i: (i, 0)),
            pl.BlockSpec((_SB, 1), lambda i: (i, 0)),
            pl.BlockSpec((_H, _H), lambda i: (0, 0)),
            pl.BlockSpec((1, _H), lambda i: (0, 0)),
            pl.BlockSpec((_H, _H), lambda i: (0, 0)),
            pl.BlockSpec((1, _H), lambda i: (0, 0)),
        ],
        out_specs=pl.BlockSpec((_SB, _H), lambda i: (i, 0)),
        out_shape=jax.ShapeDtypeStruct((_NSEG, _H), jnp.float32),
    )(acc, cnt, W3.astype(bf), b3.reshape(1, _H), W4.astype(bf),
      b4.reshape(1, _H))
    return out


# speedup vs baseline: 1.2295x; 1.2295x over previous
import functools

import jax
import jax.numpy as jnp
from jax import lax
from jax.experimental import pallas as pl
from jax.experimental.pallas import tpu as pltpu

_H = 256
_NROWS = 160000
_NSEG = 10000
_R = 3200
_W = 256
_SB = 2000

_DN = (((1,), (1,)), ((), ()))


def _phi_reduce_body(idx_ref, x_ref, w1_ref, b1_ref, w2_ref, b2_ref,
                     acc_ref, cnt_ref):
    i = pl.program_id(0)

    @pl.when(i == 0)
    def _init():
        acc_ref[...] = jnp.zeros_like(acc_ref)
        cnt_ref[...] = jnp.zeros_like(cnt_ref)

    x = x_ref[...].astype(jnp.bfloat16)
    h = jnp.maximum(
        lax.dot_general(x, w1_ref[...], _DN, preferred_element_type=jnp.float32)
        + b1_ref[...], 0.0)
    h = jnp.maximum(
        lax.dot_general(h.astype(jnp.bfloat16), w2_ref[...], _DN,
                        preferred_element_type=jnp.float32)
        + b2_ref[...], 0.0)
    hb = h.astype(jnp.bfloat16)

    idx = idx_ref[0]
    s0 = jnp.min(idx)
    max_idx = jnp.max(idx)
    s1 = jnp.minimum((s0 // 8) * 8, _NSEG - _W) + _W
    n_pass = 1 + jnp.maximum(0, (max_idx - s1 + _W) // _W)

    def pass_body(_, s):
        base = jnp.minimum((s // 8) * 8, _NSEG - _W)
        lanes = lax.broadcasted_iota(jnp.int32, (_W, _R), 0) + base
        idx_b = jnp.broadcast_to(idx, (_W, _R))
        sel = (idx_b == lanes) & (idx_b >= s)
        oh = sel.astype(jnp.bfloat16)
        contrib = jnp.dot(oh, hb, preferred_element_type=jnp.float32)
        acc_ref[pl.ds(base, _W), :] += contrib
        cnt_ref[pl.ds(base, _W), :] += jnp.sum(
            sel.astype(jnp.float32), axis=1, keepdims=True)
        return base + _W

    lax.fori_loop(0, n_pass, pass_body, s0)


def _rho_body(acc_ref, cnt_ref, w3_ref, b3_ref, w4_ref, b4_ref, out_ref):
    pooled = acc_ref[...] / jnp.maximum(cnt_ref[...], 1.0)
    o = jnp.maximum(
        lax.dot_general(pooled.astype(jnp.bfloat16), w3_ref[...], _DN,
                        preferred_element_type=jnp.float32) + b3_ref[...], 0.0)
    o = jnp.maximum(
        lax.dot_general(o.astype(jnp.bfloat16), w4_ref[...], _DN,
                        preferred_element_type=jnp.float32) + b4_ref[...], 0.0)
    out_ref[...] = o


@jax.jit
def kernel(x, molecule_idx, W1, b1, W2, b2, W3, b3, W4, b4):
    nchunks = _NROWS // _R
    idx3 = molecule_idx.astype(jnp.int32).reshape(nchunks, 1, _R)
    bf = jnp.bfloat16

    acc, cnt = pl.pallas_call(
        _phi_reduce_body,
        grid=(nchunks,),
        in_specs=[
            pl.BlockSpec((1, 1, _R), lambda i: (i, 0, 0)),
            pl.BlockSpec((_R, _H), lambda i: (i, 0)),
            pl.BlockSpec((_H, _H), lambda i: (0, 0)),
            pl.BlockSpec((1, _H), lambda i: (0, 0)),
            pl.BlockSpec((_H, _H), lambda i: (0, 0)),
            pl.BlockSpec((1, _H), lambda i: (0, 0)),
        ],
        out_specs=[
            pl.BlockSpec((_NSEG, _H), lambda i: (0, 0)),
            pl.BlockSpec((_NSEG, 1), lambda i: (0, 0)),
        ],
        out_shape=[
            jax.ShapeDtypeStruct((_NSEG, _H), jnp.float32),
            jax.ShapeDtypeStruct((_NSEG, 1), jnp.float32),
        ],
    )(idx3, x, W1.astype(bf), b1.reshape(1, _H),
      W2.astype(bf), b2.reshape(1, _H))

    out = pl.pallas_call(
        _rho_body,
        grid=(_NSEG // _SB,),
        in_specs=[
            pl.BlockSpec((_SB, _H), lambda i: (i, 0)),
            pl.BlockSpec((_SB, 1), lambda i: (i, 0)),
            pl.BlockSpec((_H, _H), lambda i: (0, 0)),
            pl.BlockSpec((1, _H), lambda i: (0, 0)),
            pl.BlockSpec((_H, _H), lambda i: (0, 0)),
            pl.BlockSpec((1, _H), lambda i: (0, 0)),
        ],
        out_specs=pl.BlockSpec((_SB, _H), lambda i: (i, 0)),
        out_shape=jax.ShapeDtypeStruct((_NSEG, _H), jnp.float32),
    )(acc, cnt, W3.astype(bf), b3.reshape(1, _H), W4.astype(bf),
      b4.reshape(1, _H))
    return out
